# baseline (speedup 1.0000x reference)
import jax
import jax.numpy as jnp
from jax.experimental import pallas as pl
from jax.experimental.pallas import tpu as pltpu

_C = 2048
_G = 128
_H = 64
_CH = 512
_KB = 512
_NEG = -1e30
_SCALE = 0.125


def _dot(a, b):
    return jax.lax.dot(a, b, preferred_element_type=jnp.float32)


def _dot_t(a, b):
    return jax.lax.dot_general(a, b, (((1,), (1,)), ((), ())),
                               preferred_element_type=jnp.float32)


def _tl_kernel(m_ref, ddi_ref, unr_ref, h0_ref, mrow_ref,
               ddi_c_ref, unr_c_ref, m2c_ref, m3c_ref, n2_ref, n3_ref,
               wir_ref, wiz_ref, win_ref, whr_ref, whz_ref, whn_ref,
               bir_ref, biz_ref, bin_ref, bhr_ref, bhz_ref, bhn_ref,
               wq_ref, bq_ref, wk_ref, bk_ref, wv_ref, bv_ref,
               out_ref, hnew_ref):
    f32 = jnp.float32
    bf16 = jnp.bfloat16
    mm = m_ref[:]
    h0 = h0_ref[:]

    r = jax.nn.sigmoid(_dot(mm, wir_ref[:]) + bir_ref[:]
                       + _dot(h0, whr_ref[:]) + bhr_ref[:])
    z = jax.nn.sigmoid(_dot(mm, wiz_ref[:]) + biz_ref[:]
                       + _dot(h0, whz_ref[:]) + bhz_ref[:])
    n = jnp.tanh(_dot(mm, win_ref[:]) + bin_ref[:]
                 + r * (_dot(h0, whn_ref[:]) + bhn_ref[:]))
    h_m1 = (1.0 - z) * n + z * h0

    qd = (_dot(ddi_ref[:], wq_ref[:]) + bq_ref[:]) * _SCALE
    qu = (_dot(unr_ref[:], wq_ref[:]) + bq_ref[:]) * _SCALE

    n2 = n2_ref[0, 0]
    n3 = n3_ref[0, 0]
    row_iota = jax.lax.broadcasted_iota(jnp.int32, (_C, 1), 0)
    bias2 = jnp.where(row_iota < n2, 0.0, _NEG).astype(f32)
    bias3 = jnp.where(row_iota < n3, 0.0, _NEG).astype(f32)
    kd = _dot(ddi_c_ref[:], wk_ref[:]) + bk_ref[:]
    ku = _dot(unr_c_ref[:], wk_ref[:]) + bk_ref[:]
    vd = _dot(m2c_ref[:], wv_ref[:]) + bv_ref[:]
    vu = _dot(m3c_ref[:], wv_ref[:]) + bv_ref[:]
    ones_c = jnp.ones((_C, 1), f32)
    kdh = jnp.concatenate([kd, bias2], axis=1).astype(bf16)
    kuh = jnp.concatenate([ku, bias3], axis=1).astype(bf16)
    vdh = jnp.concatenate([vd, ones_c], axis=1).astype(bf16)
    vuh = jnp.concatenate([vu, ones_c], axis=1).astype(bf16)

    nkb = _C // _KB

    def att_chunk(x):
        xh = jnp.concatenate(
            [x, jnp.ones((_CH, 1), f32)], axis=1).astype(bf16)
        o = jnp.zeros((_CH, _H + 1), f32)
        mx = jnp.full((_CH, 1), _NEG, f32)

        def block(kh, vh, cnt, o, mx):
            for kb in range(nkb):
                def do(o=o, mx=mx, kb=kb):
                    l = _dot_t(xh, kh[kb * _KB:(kb + 1) * _KB, :])
                    nmx = jnp.maximum(mx, jnp.max(l, axis=1, keepdims=True))
                    alpha = jnp.exp(mx - nmx)
                    e = jnp.exp(l - nmx).astype(bf16)
                    return (o * alpha
                            + _dot(e, vh[kb * _KB:(kb + 1) * _KB, :]), nmx)
                o, mx = jax.lax.cond(kb * _KB < cnt, do, lambda: (o, mx))
            return o, mx

        o, mx = block(kdh, vdh, n2, o, mx)
        o, mx = block(kuh, vuh, n3, o, mx)
        return jnp.tanh(o[:, 0:_H] / o[:, _H:_H + 1])

    h2 = jnp.concatenate(
        [att_chunk(qd[c * _CH:(c + 1) * _CH, :]) for c in range(_C // _CH)],
        axis=0)
    h3 = jnp.concatenate(
        [att_chunk(qu[c * _CH:(c + 1) * _CH, :]) for c in range(_C // _CH)],
        axis=0)

    m1r = mrow_ref[:, 0:1]
    m2r = mrow_ref[:, 1:2]
    m3r = mrow_ref[:, 2:3]

    hnew = jnp.where(m1r > 0, h_m1, 0.0)
    hnew = jnp.where(m2r > 0, h2, hnew)
    hnew = jnp.where(m3r > 0, h3, hnew)
    hnew_ref[:] = hnew

    o1 = jnp.max(jnp.where(m1r > 0, h_m1, _NEG), axis=0, keepdims=True)
    o2 = jnp.max(jnp.where(m2r > 0, h2, _NEG), axis=0, keepdims=True)
    o3 = jnp.max(jnp.where(m3r > 0, h3, _NEG), axis=0, keepdims=True)
    out_ref[:] = jnp.maximum(o1, jnp.maximum(o2, o3))


def kernel(m_embeddings, divided, ddi_embeddings, unrelated_embeddings,
           hidden_state, W_ih, b_ih, W_hh, b_hh, Wq, bq, Wk, bk, Wv, bv):
    f32 = jnp.float32
    i32 = jnp.int32
    mrow = (divided > 0).astype(f32)
    mask2 = divided[:, 1] > 0
    mask3 = divided[:, 2] > 0
    idx2 = jnp.nonzero(mask2, size=_C, fill_value=0)[0]
    idx3 = jnp.nonzero(mask3, size=_C, fill_value=0)[0]
    n2 = jnp.sum(mask2, dtype=i32).reshape(1, 1)
    n3 = jnp.sum(mask3, dtype=i32).reshape(1, 1)
    ddi_c = jnp.take(ddi_embeddings, idx2, axis=0)
    unr_c = jnp.take(unrelated_embeddings, idx3, axis=0)
    m2_c = jnp.take(m_embeddings, idx2, axis=0)
    m3_c = jnp.take(m_embeddings, idx3, axis=0)

    wih_t = W_ih.T
    whh_t = W_hh.T
    wir, wiz, win = (wih_t[:, :_H], wih_t[:, _H:2 * _H], wih_t[:, 2 * _H:])
    whr, whz, whn = (whh_t[:, :_H], whh_t[:, _H:2 * _H], whh_t[:, 2 * _H:])
    bir, biz, bin_ = (b_ih[None, :_H], b_ih[None, _H:2 * _H],
                      b_ih[None, 2 * _H:])
    bhr, bhz, bhn = (b_hh[None, :_H], b_hh[None, _H:2 * _H],
                     b_hh[None, 2 * _H:])

    out, h_new = pl.pallas_call(
        _tl_kernel,
        out_shape=(jax.ShapeDtypeStruct((1, _H), f32),
                   jax.ShapeDtypeStruct((_C, _H), f32)),
        compiler_params=pltpu.CompilerParams(
            vmem_limit_bytes=112 * 1024 * 1024),
    )(m_embeddings, ddi_embeddings, unrelated_embeddings, hidden_state,
      mrow,
      ddi_c, unr_c, m2_c, m3_c, n2, n3,
      wir, wiz, win, whr, whz, whn,
      bir, biz, bin_, bhr, bhz, bhn,
      Wq.T, bq[None, :], Wk.T, bk[None, :], Wv.T, bv[None, :])

    return out.reshape(_H), h_new


# speedup vs baseline: 5.7706x; 5.7706x over previous
import jax
import jax.numpy as jnp
from jax.experimental import pallas as pl
from jax.experimental.pallas import tpu as pltpu

_C = 2048
_G = 128
_H = 64
_CH = 512
_NEG = -1e30
_SCALE = 0.125


def _dot(a, b):
    return jax.lax.dot(a, b, preferred_element_type=jnp.float32)


def _dot_t(a, b):
    return jax.lax.dot_general(a, b, (((1,), (1,)), ((), ())),
                               preferred_element_type=jnp.float32)


def _tl_kernel(m_ref, ddi_ref, unr_ref, h0_ref, mrow_ref,
               wir_ref, wiz_ref, win_ref, whr_ref, whz_ref, whn_ref,
               bir_ref, biz_ref, bin_ref, bhr_ref, bhz_ref, bhn_ref,
               wq_ref, bq_ref, wk_ref, bk_ref, wv_ref, bv_ref,
               out_ref, hnew_ref):
    bf16 = jnp.bfloat16
    mm = m_ref[:]
    h0 = h0_ref[:]

    r = jax.nn.sigmoid(_dot(mm, wir_ref[:]) + bir_ref[:]
                       + _dot(h0, whr_ref[:]) + bhr_ref[:])
    z = jax.nn.sigmoid(_dot(mm, wiz_ref[:]) + biz_ref[:]
                       + _dot(h0, whz_ref[:]) + bhz_ref[:])
    n = jnp.tanh(_dot(mm, win_ref[:]) + bin_ref[:]
                 + r * (_dot(h0, whn_ref[:]) + bhn_ref[:]))
    h_m1 = (1.0 - z) * n + z * h0

    ddi = ddi_ref[:]
    unr = unr_ref[:]
    qd = (_dot(ddi, wq_ref[:]) + bq_ref[:]) * _SCALE
    qu = (_dot(unr, wq_ref[:]) + bq_ref[:]) * _SCALE
    kd = _dot(ddi, wk_ref[:]) + bk_ref[:]
    ku = _dot(unr, wk_ref[:]) + bk_ref[:]
    val = _dot(mm, wv_ref[:]) + bv_ref[:]

    b2r = (mrow_ref[:, 1:2] - 1.0) * 1e30
    b3r = (mrow_ref[:, 2:3] - 1.0) * 1e30
    kdh = jnp.concatenate([kd, b2r], axis=1).astype(bf16)
    kuh = jnp.concatenate([ku, b3r], axis=1).astype(bf16)
    valh = jnp.concatenate(
        [val, jnp.ones((_C, 1), jnp.float32)], axis=1).astype(bf16)

    def att_chunk(x):
        xh = jnp.concatenate(
            [x, jnp.ones((_CH, 1), jnp.float32)], axis=1).astype(bf16)
        ld = _dot_t(xh, kdh)
        lu = _dot_t(xh, kuh)
        mx = jnp.maximum(jnp.max(ld, axis=1, keepdims=True),
                         jnp.max(lu, axis=1, keepdims=True))
        ed = jnp.exp(ld - mx).astype(bf16)
        eu = jnp.exp(lu - mx).astype(bf16)
        o = _dot(ed, valh) + _dot(eu, valh)
        return jnp.tanh(o[:, 0:_H] / o[:, _H:_H + 1])

    h2 = jnp.concatenate(
        [att_chunk(qd[c * _CH:(c + 1) * _CH, :]) for c in range(_C // _CH)],
        axis=0)
    h3 = jnp.concatenate(
        [att_chunk(qu[c * _CH:(c + 1) * _CH, :]) for c in range(_C // _CH)],
        axis=0)

    m1r = mrow_ref[:, 0:1]
    m2r = mrow_ref[:, 1:2]
    m3r = mrow_ref[:, 2:3]

    hnew = jnp.where(m1r > 0, h_m1, 0.0)
    hnew = jnp.where(m2r > 0, h2, hnew)
    hnew = jnp.where(m3r > 0, h3, hnew)
    hnew_ref[:] = hnew

    o1 = jnp.max(jnp.where(m1r > 0, h_m1, _NEG), axis=0, keepdims=True)
    o2 = jnp.max(jnp.where(m2r > 0, h2, _NEG), axis=0, keepdims=True)
    o3 = jnp.max(jnp.where(m3r > 0, h3, _NEG), axis=0, keepdims=True)
    out_ref[:] = jnp.maximum(o1, jnp.maximum(o2, o3))


def kernel(m_embeddings, divided, ddi_embeddings, unrelated_embeddings,
           hidden_state, W_ih, b_ih, W_hh, b_hh, Wq, bq, Wk, bk, Wv, bv):
    f32 = jnp.float32
    mrow = (divided > 0).astype(f32)

    wih_t = W_ih.T
    whh_t = W_hh.T
    wir, wiz, win = (wih_t[:, :_H], wih_t[:, _H:2 * _H], wih_t[:, 2 * _H:])
    whr, whz, whn = (whh_t[:, :_H], whh_t[:, _H:2 * _H], whh_t[:, 2 * _H:])
    bir, biz, bin_ = (b_ih[None, :_H], b_ih[None, _H:2 * _H],
                      b_ih[None, 2 * _H:])
    bhr, bhz, bhn = (b_hh[None, :_H], b_hh[None, _H:2 * _H],
                     b_hh[None, 2 * _H:])

    out, h_new = pl.pallas_call(
        _tl_kernel,
        out_shape=(jax.ShapeDtypeStruct((1, _H), f32),
                   jax.ShapeDtypeStruct((_C, _H), f32)),
        compiler_params=pltpu.CompilerParams(
            vmem_limit_bytes=112 * 1024 * 1024),
    )(m_embeddings, ddi_embeddings, unrelated_embeddings, hidden_state,
      mrow,
      wir, wiz, win, whr, whz, whn,
      bir, biz, bin_, bhr, bhz, bhn,
      Wq.T, bq[None, :], Wk.T, bk[None, :], Wv.T, bv[None, :])

    return out.reshape(_H), h_new
